# baseline (speedup 1.0000x reference)
import functools

import numpy as np
import jax
import jax.numpy as jnp
from jax import lax
from jax.experimental import pallas as pl
from jax.experimental.pallas import tpu as pltpu
from jax.experimental.pallas import tpu_sc as plsc

_B, _F, _L, _D, _V = 4096, 26, 4, 32, 100000
_BF = _B * _F

_NW = 32
_RPW = _BF // _NW
_CH = 256
_NCH = _RPW // _CH
_GSUB = 128
_NSUB = _CH * _L // _GSUB


def _sc_pool_body(tbl_hbm, idx_hbm, out_hbm, idx_v, rows_v, out_v, sem):
    wid = lax.axis_index("s") * 2 + lax.axis_index("c")

    def chunk(ci, carry):
        base = wid * _RPW + ci * _CH
        ib = wid * (_RPW // _GSUB * _L) + ci * _NSUB
        pltpu.sync_copy(idx_hbm.at[pl.ds(ib, _NSUB)], idx_v)
        cps = [
            pltpu.async_copy(
                tbl_hbm.at[idx_v.at[j]],
                rows_v.at[pl.ds(j * _GSUB, _GSUB)],
                sem,
            )
            for j in range(_NSUB)
        ]
        for c in cps:
            c.wait()

        def pool(i, c2):
            for h in range(2):
                s = pl.ds(h * 16, 16)
                out_v[i, s] = (rows_v[4 * i, s] + rows_v[4 * i + 1, s]) + (
                    rows_v[4 * i + 2, s] + rows_v[4 * i + 3, s]
                )
            return c2

        lax.fori_loop(0, _CH, pool, 0)
        pltpu.sync_copy(out_v, out_hbm.at[pl.ds(base, _CH)])
        return carry

    lax.fori_loop(0, _NCH, chunk, 0)


@functools.cache
def _sc_pool():
    return pl.kernel(
        _sc_pool_body,
        out_type=jax.ShapeDtypeStruct((_BF, _D), jnp.float32),
        mesh=plsc.VectorSubcoreMesh(core_axis_name="c", subcore_axis_name="s"),
        scratch_types=[
            pltpu.VMEM((_NSUB, _GSUB), jnp.int32),
            pltpu.VMEM((_CH * _L, _D), jnp.float32),
            pltpu.VMEM((_CH, _D), jnp.float32),
            pltpu.SemaphoreType.DMA,
        ],
        compiler_params=pltpu.CompilerParams(use_tc_tiling_on_sc=False),
    )

_BB = 256
_NBLK = _B // _BB
_NF = _F + 1
_NPAIR = _NF * _NF
_IU0, _IU1 = np.triu_indices(_NF, k=1)
_PAIR_POS = (_IU0 * _NF + _IU1).astype(np.int32)


def _tc_body(dx, sp, bw0, bb0, bw1, bb1, bw2, bb2,
             tw0h, tw0f, tb0, tw1, tb1, tw2, tb2, tw3, tb3, tw4, tb4, out):
    f32 = jnp.float32
    x = dx[...]
    h = jnp.maximum(jnp.dot(x, bw0[...], preferred_element_type=f32) + bb0[...], 0.0)
    h = jnp.maximum(jnp.dot(h, bw1[...], preferred_element_type=f32) + bb1[...], 0.0)
    h = jnp.dot(h, bw2[...], preferred_element_type=f32) + bb2[...]

    comb = jnp.concatenate([sp[...], h], axis=1)
    c3 = comb.reshape(_BB, _NF, _D)
    inter = lax.dot_general(
        c3, c3, (((2,), (2,)), ((0,), (0,))), preferred_element_type=f32
    )
    f729 = inter.reshape(_BB, _NPAIR)

    t = jnp.dot(h, tw0h[...], preferred_element_type=f32)
    t = t + jnp.dot(f729, tw0f[...], preferred_element_type=f32)
    t = jnp.maximum(t + tb0[...], 0.0)
    t = jnp.maximum(jnp.dot(t, tw1[...], preferred_element_type=f32) + tb1[...], 0.0)
    t = jnp.maximum(jnp.dot(t, tw2[...], preferred_element_type=f32) + tb2[...], 0.0)
    t = jnp.maximum(jnp.dot(t, tw3[...], preferred_element_type=f32) + tb3[...], 0.0)
    out[...] = jnp.dot(t, tw4[...], preferred_element_type=f32) + tb4[...]


def _full(shape):
    return pl.BlockSpec(shape, lambda i: (0, 0))


def _tc_call(dense_x, sp, bw0, bb0, bw1, bb1, bw2, bb2,
             tw0h, tw0f, tb0, tw1, tb1, tw2, tb2, tw3, tb3, tw4, tb4):
    weights = (bw0, bb0, bw1, bb1, bw2, bb2,
               tw0h, tw0f, tb0, tw1, tb1, tw2, tb2, tw3, tb3, tw4, tb4)
    in_specs = [
        pl.BlockSpec((_BB, 13), lambda i: (i, 0)),
        pl.BlockSpec((_BB, _F * _D), lambda i: (i, 0)),
    ] + [_full(w.shape) for w in weights]
    return pl.pallas_call(
        _tc_body,
        grid=(_NBLK,),
        in_specs=in_specs,
        out_specs=pl.BlockSpec((_BB, 1), lambda i: (i, 0)),
        out_shape=jax.ShapeDtypeStruct((_B, 1), jnp.float32),
    )(dense_x, sp, *weights)


def kernel(dense_x, kjt, tables, bw0, bb0, bw1, bb1, bw2, bb2,
           tw0, tb0, tw1, tb1, tw2, tb2, tw3, tb3, tw4, tb4):
    offs = jnp.repeat(jnp.arange(_F, dtype=jnp.int32) * _V, _L)[None, :]
    idx = (kjt.reshape(_B, _F * _L) + offs).reshape(_BF * _L // _GSUB, _GSUB)
    pooled = _sc_pool()(tables.reshape(_F * _V, _D), idx)
    sp = pooled.reshape(_B, _F * _D)

    tw0h = tw0[:_D]
    tw0f = jnp.zeros((_NPAIR, tw0.shape[1]), tw0.dtype).at[_PAIR_POS].set(tw0[_D:])
    b2 = lambda b: b.reshape(1, -1)
    return _tc_call(dense_x, sp, bw0, b2(bb0), bw1, b2(bb1), bw2, b2(bb2),
                    tw0h, tw0f, b2(tb0), tw1, b2(tb1), tw2, b2(tb2),
                    tw3, b2(tb3), tw4, b2(tb4))


# speedup vs baseline: 1.6846x; 1.6846x over previous
import functools

import numpy as np
import jax
import jax.numpy as jnp
from jax import lax
from jax.experimental import pallas as pl
from jax.experimental.pallas import tpu as pltpu
from jax.experimental.pallas import tpu_sc as plsc

_B, _F, _L, _D, _V = 4096, 26, 4, 32, 100000
_BF = _B * _F

_NW = 32


def _sc_pool_body(tbl_hbm, kidx_hbm, out_hbm, plane_v, idx_v, out_v, sem, sem2):
    d = lax.axis_index("s") * 2 + lax.axis_index("c")

    def task(f, carry):
        cp = pltpu.async_copy(tbl_hbm.at[f, d], plane_v, sem)
        ci = pltpu.async_copy(kidx_hbm.at[f], idx_v, sem2)
        cp.wait()
        ci.wait()

        def pool(vb, c2):
            s = pl.ds(vb * 16, 16)
            acc = plsc.load_gather(plane_v, [idx_v[0, s]])
            acc = acc + plsc.load_gather(plane_v, [idx_v[1, s]])
            acc = acc + plsc.load_gather(plane_v, [idx_v[2, s]])
            acc = acc + plsc.load_gather(plane_v, [idx_v[3, s]])
            out_v[s] = acc
            return c2

        lax.fori_loop(0, _B // 16, pool, 0)
        pltpu.sync_copy(out_v, out_hbm.at[f, d])
        return carry

    lax.fori_loop(0, _F, task, 0)


@functools.cache
def _sc_pool():
    return pl.kernel(
        _sc_pool_body,
        out_type=jax.ShapeDtypeStruct((_F, _D, _B), jnp.float32),
        mesh=plsc.VectorSubcoreMesh(core_axis_name="c", subcore_axis_name="s"),
        scratch_types=[
            pltpu.VMEM((_V,), jnp.float32),
            pltpu.VMEM((_L, _B), jnp.int32),
            pltpu.VMEM((_B,), jnp.float32),
            pltpu.SemaphoreType.DMA,
            pltpu.SemaphoreType.DMA,
        ],
        compiler_params=pltpu.CompilerParams(
            use_tc_tiling_on_sc=False, needs_layout_passes=False
        ),
    )


_BB = 256
_NBLK = _B // _BB
_NF = _F + 1
_NPAIR = _NF * _NF
_IU0, _IU1 = np.triu_indices(_NF, k=1)
_PAIR_POS = (_IU0 * _NF + _IU1).astype(np.int32)


def _tc_body(dx, sp, bw0, bb0, bw1, bb1, bw2, bb2,
             tw0h, tw0f, tb0, tw1, tb1, tw2, tb2, tw3, tb3, tw4, tb4, out):
    f32 = jnp.float32
    x = dx[...]
    h = jnp.maximum(jnp.dot(x, bw0[...], preferred_element_type=f32) + bb0[...], 0.0)
    h = jnp.maximum(jnp.dot(h, bw1[...], preferred_element_type=f32) + bb1[...], 0.0)
    h = jnp.dot(h, bw2[...], preferred_element_type=f32) + bb2[...]

    comb = jnp.concatenate([sp[...], h], axis=1)
    c3 = comb.reshape(_BB, _NF, _D)
    inter = lax.dot_general(
        c3, c3, (((2,), (2,)), ((0,), (0,))), preferred_element_type=f32
    )
    f729 = inter.reshape(_BB, _NPAIR)

    t = jnp.dot(h, tw0h[...], preferred_element_type=f32)
    t = t + jnp.dot(f729, tw0f[...], preferred_element_type=f32)
    t = jnp.maximum(t + tb0[...], 0.0)
    t = jnp.maximum(jnp.dot(t, tw1[...], preferred_element_type=f32) + tb1[...], 0.0)
    t = jnp.maximum(jnp.dot(t, tw2[...], preferred_element_type=f32) + tb2[...], 0.0)
    t = jnp.maximum(jnp.dot(t, tw3[...], preferred_element_type=f32) + tb3[...], 0.0)
    out[...] = jnp.dot(t, tw4[...], preferred_element_type=f32) + tb4[...]


def _full(shape):
    return pl.BlockSpec(shape, lambda i: (0, 0))


def _tc_call(dense_x, sp, bw0, bb0, bw1, bb1, bw2, bb2,
             tw0h, tw0f, tb0, tw1, tb1, tw2, tb2, tw3, tb3, tw4, tb4):
    weights = (bw0, bb0, bw1, bb1, bw2, bb2,
               tw0h, tw0f, tb0, tw1, tb1, tw2, tb2, tw3, tb3, tw4, tb4)
    in_specs = [
        pl.BlockSpec((_BB, 13), lambda i: (i, 0)),
        pl.BlockSpec((_BB, _F * _D), lambda i: (i, 0)),
    ] + [_full(w.shape) for w in weights]
    return pl.pallas_call(
        _tc_body,
        grid=(_NBLK,),
        in_specs=in_specs,
        out_specs=pl.BlockSpec((_BB, 1), lambda i: (i, 0)),
        out_shape=jax.ShapeDtypeStruct((_B, 1), jnp.float32),
    )(dense_x, sp, *weights)


def kernel(dense_x, kjt, tables, bw0, bb0, bw1, bb1, bw2, bb2,
           tw0, tb0, tw1, tb1, tw2, tb2, tw3, tb3, tw4, tb4):
    tbl_t = jnp.transpose(tables, (0, 2, 1))
    kidx = jnp.transpose(kjt, (1, 2, 0))
    pooled = _sc_pool()(tbl_t, kidx)
    sp = jnp.transpose(pooled, (2, 0, 1)).reshape(_B, _F * _D)

    tw0h = tw0[:_D]
    tw0f = jnp.zeros((_NPAIR, tw0.shape[1]), tw0.dtype).at[_PAIR_POS].set(tw0[_D:])
    b2 = lambda b: b.reshape(1, -1)
    return _tc_call(dense_x, sp, bw0, b2(bb0), bw1, b2(bb1), bw2, b2(bb2),
                    tw0h, tw0f, b2(tb0), tw1, b2(tb1), tw2, b2(tb2),
                    tw3, b2(tb3), tw4, b2(tb4))
